# baseline (speedup 1.0000x reference)
import functools

import jax
import jax.numpy as jnp
import numpy as np
from jax import lax
from jax.experimental import pallas as pl
from jax.experimental.pallas import tpu as pltpu
from jax.experimental.pallas import tpu_sc as plsc

D = 256
H = 8
HD = 32
NBQ = 2
NP = 4
GH, GW = 100, 100
NQ = GH * GW
RPQ = NBQ * NP * 4

QB = 400
NQB = NQ // QB

NW = 32
CQ = 8
ROWSC = CQ * D
NCHUNK = NQ // CQ
NCHT = 41

_PERM_SO = np.empty(128, np.int64)
_PERM_AW = np.empty(64, np.int64)
for _h in range(H):
    for _b in range(NBQ):
        for _p in range(NP):
            _s = _p * 16 + _h * 2 + _b
            _PERM_SO[_s] = _h * 16 + _b * 8 + _p * 2 + 0
            _PERM_SO[64 + _s] = _h * 16 + _b * 8 + _p * 2 + 1
            _PERM_AW[_s] = _h * 8 + _b * 4 + _p
_BSEL = np.arange(64) % 2

_S01 = np.zeros((64, 256), np.float32)
_S23 = np.zeros((64, 256), np.float32)
_SCW = [np.zeros((64, 256), np.float32) for _ in range(4)]
_BIAS_IDX = np.zeros((1, 256), np.float32)
for _t in range(256):
    _h = _t // 32
    _m = _t % 32
    _bp = _m // 4
    _c = _m % 4
    _b = _bp // 4
    _p = _bp % 4
    _s = _p * 16 + _h * 2 + _b
    if _c in (0, 1):
        _S01[_s, _t] = 1.0
    else:
        _S23[_s, _t] = 1.0
    _SCW[_c][_s, _t] = 1.0
    _BIAS_IDX[0, _t] = _h + (8.0 if _c in (1, 3) else 0.0)

_PERM_WOUT = np.empty(256, np.int64)
for _h in range(H):
    for _j in range(32):
        _ch = 2 * _j if _j < 16 else 2 * (_j - 16) + 1
        _PERM_WOUT[_h * 32 + _j] = _h * 32 + _ch


def _tc1_body(q_ref, qp_ref, xb_ref, yb_ref, Wv_ref, bv_ref, Wso_ref,
              bso_ref, Waw_ref, baw_ref, S01_ref, S23_ref, SC0_ref, SC1_ref,
              SC2_ref, SC3_ref, bi_ref, v_ref, idxL_ref, idxR_ref,
              wL_ref, wR_ref):
    q = q_ref[...]
    v_ref[...] = (jnp.dot(q, Wv_ref[...], preferred_element_type=jnp.float32)
                  + bv_ref[...]).astype(jnp.bfloat16)
    qc = jnp.concatenate([q, q + qp_ref[...]], axis=-1)
    so = jnp.dot(qc, Wso_ref[...], preferred_element_type=jnp.float32) + bso_ref[...]
    awl = jnp.dot(qc, Waw_ref[...], preferred_element_type=jnp.float32) + baw_ref[...]
    m = jnp.maximum(jnp.maximum(awl[:, 0:16], awl[:, 16:32]),
                    jnp.maximum(awl[:, 32:48], awl[:, 48:64]))
    e = jnp.exp(awl - jnp.concatenate([m, m, m, m], axis=-1))
    s = e[:, 0:16] + e[:, 16:32] + e[:, 32:48] + e[:, 48:64]
    aw = e / jnp.concatenate([s, s, s, s], axis=-1)

    xf = so[:, 0:64] + xb_ref[...]
    yf = so[:, 64:128] + yb_ref[...]
    x0 = jnp.floor(xf)
    fx = xf - x0
    cb = jnp.clip(x0, 0.0, 98.0)
    wc0 = jnp.where(x0 == cb, 1.0 - fx, jnp.where(x0 + 1.0 == cb, fx, 0.0))
    wc1 = jnp.where(x0 == cb, fx, jnp.where(x0 == cb + 1.0, 1.0 - fx, 0.0))
    y0 = jnp.floor(yf)
    fy = yf - y0
    rb0 = jnp.clip(y0, 0.0, 99.0)
    rb1 = jnp.clip(y0 + 1.0, 0.0, 99.0)
    wr0 = jnp.where(y0 == rb0, 1.0 - fy, 0.0)
    wr1 = jnp.where(y0 + 1.0 == rb1, fy, 0.0)
    idxA = rb0 * 100.0 + cb
    idxB = rb1 * 100.0 + cb
    idx256 = (jnp.dot(idxA, S01_ref[...], preferred_element_type=jnp.float32,
                      precision=lax.Precision.HIGHEST)
              + jnp.dot(idxB, S23_ref[...], preferred_element_type=jnp.float32,
                        precision=lax.Precision.HIGHEST)) * 8.0 + bi_ref[...]
    idx256 = (idx256 + 0.5).astype(jnp.int32)
    idxL_ref[...] = idx256[:, 0:128]
    idxR_ref[...] = idx256[:, 128:256]
    half = 0.5 * aw
    w256 = (
        jnp.dot(half * wr0 * wc0, SC0_ref[...], preferred_element_type=jnp.float32)
        + jnp.dot(half * wr0 * wc1, SC1_ref[...], preferred_element_type=jnp.float32)
        + jnp.dot(half * wr1 * wc0, SC2_ref[...], preferred_element_type=jnp.float32)
        + jnp.dot(half * wr1 * wc1, SC3_ref[...], preferred_element_type=jnp.float32))
    wL_ref[...] = w256[:, 0:128]
    wR_ref[...] = w256[:, 128:256]


def _tc1_call(q, qp, xb, yb, Wv, bv, Wso, bso, Waw, baw, interpret=False):
    consts = [jnp.asarray(a) for a in
              (_S01, _S23, *_SCW, _BIAS_IDX)]
    cspecs = [pl.BlockSpec((64, 256), lambda i: (0, 0))] * 6 + [
        pl.BlockSpec((1, 256), lambda i: (0, 0))]
    h128 = pl.BlockSpec((QB, 128), lambda i: (i, 0))
    return pl.pallas_call(
        _tc1_body,
        grid=(NQB,),
        in_specs=[
            pl.BlockSpec((QB, D), lambda i: (i, 0)),
            pl.BlockSpec((QB, D), lambda i: (i, 0)),
            pl.BlockSpec((QB, 64), lambda i: (i, 0)),
            pl.BlockSpec((QB, 64), lambda i: (i, 0)),
            pl.BlockSpec((D, D), lambda i: (0, 0)),
            pl.BlockSpec((1, D), lambda i: (0, 0)),
            pl.BlockSpec((2 * D, 128), lambda i: (0, 0)),
            pl.BlockSpec((1, 128), lambda i: (0, 0)),
            pl.BlockSpec((2 * D, 64), lambda i: (0, 0)),
            pl.BlockSpec((1, 64), lambda i: (0, 0)),
        ] + cspecs,
        out_specs=[
            pl.BlockSpec((QB, D), lambda i: (i, 0)),
            h128, h128, h128, h128,
        ],
        out_shape=[
            jax.ShapeDtypeStruct((NQ, D), jnp.bfloat16),
            jax.ShapeDtypeStruct((NQ, 128), jnp.int32),
            jax.ShapeDtypeStruct((NQ, 128), jnp.int32),
            jax.ShapeDtypeStruct((NQ, 128), jnp.float32),
            jax.ShapeDtypeStruct((NQ, 128), jnp.float32),
        ],
        interpret=interpret,
    )(q, qp, xb, yb, Wv, bv, Wso, bso, Waw, baw, *consts)


def _splat(vec, lane):
    return lax.gather(
        vec, jnp.full((16, 1), lane, jnp.int32),
        dimension_numbers=lax.GatherDimensionNumbers(
            offset_dims=(), collapsed_slice_dims=(0,), start_index_map=(0,)),
        slice_sizes=(1,),
        mode=lax.GatherScatterMode.PROMISE_IN_BOUNDS)


def _sc_body(vt_hbm, idxL_hbm, idxR_hbm, wL_hbm, wR_hbm, outL_hbm, outR_hbm,
             iL0, iL1, iL2, iR0, iR1, iR2, wl0, wl1, wl2, wr0, wr1, wr2,
             r0, r1, r2, outL_v, outR_v,
             sio0, sio1, sio2, sg0, sg1, sg2):
    idxLs = (iL0, iL1, iL2)
    idxRs = (iR0, iR1, iR2)
    wLs = (wl0, wl1, wl2)
    wRs = (wr0, wr1, wr2)
    rows = (r0, r1, r2)
    sio = (sio0, sio1, sio2)
    sg = (sg0, sg1, sg2)
    wid = lax.axis_index("s") * 2 + lax.axis_index("c")
    c0 = (625 * wid) // 16

    def chunk_q(c):
        return jnp.minimum(c0 + c, NCHUNK - 1) * CQ

    def start_io(c, b):
        qs = chunk_q(c)
        pltpu.async_copy(idxL_hbm.at[pl.ds(qs, CQ), :], idxLs[b], sio[b])
        pltpu.async_copy(idxR_hbm.at[pl.ds(qs, CQ), :], idxRs[b], sio[b])
        pltpu.async_copy(wL_hbm.at[pl.ds(qs, CQ), :], wLs[b], sio[b])
        pltpu.async_copy(wR_hbm.at[pl.ds(qs, CQ), :], wRs[b], sio[b])

    def wait_io(b):
        pltpu.make_async_copy(idxL_hbm.at[pl.ds(0, CQ), :], idxLs[b], sio[b]).wait()
        pltpu.make_async_copy(idxR_hbm.at[pl.ds(0, CQ), :], idxRs[b], sio[b]).wait()
        pltpu.make_async_copy(wL_hbm.at[pl.ds(0, CQ), :], wLs[b], sio[b]).wait()
        pltpu.make_async_copy(wR_hbm.at[pl.ds(0, CQ), :], wRs[b], sio[b]).wait()

    def fire_g(b):
        for qi in range(CQ):
            pltpu.async_copy(
                vt_hbm.at[idxLs[b].at[qi, :]],
                rows[b].at[pl.ds(qi * 2 * 128, 128)], sg[b])
            pltpu.async_copy(
                vt_hbm.at[idxRs[b].at[qi, :]],
                rows[b].at[pl.ds((qi * 2 + 1) * 128, 128)], sg[b])

    def wait_g(b):
        for qi in range(CQ):
            pltpu.make_async_copy(
                vt_hbm.at[idxLs[b].at[qi, :]],
                rows[b].at[pl.ds(qi * 2 * 128, 128)], sg[b]).wait()
            pltpu.make_async_copy(
                vt_hbm.at[idxRs[b].at[qi, :]],
                rows[b].at[pl.ds((qi * 2 + 1) * 128, 128)], sg[b]).wait()

    def compute(c, b):
        rr = rows[b]
        wl = wLs[b]
        wr = wRs[b]

        def q_loop(qi, c2):
            base = qi * D
            for h in range(H):
                wref = wl if h < 4 else wr
                hh = h % 4
                wv0 = wref[qi, pl.ds(hh * HD, 16)]
                wv1 = wref[qi, pl.ds(hh * HD + 16, 16)]
                acc0 = jnp.zeros((16,), jnp.float32)
                acc1 = jnp.zeros((16,), jnp.float32)
                for k in range(RPQ):
                    wsp = _splat(wv0 if k < 16 else wv1, k % 16)
                    ri = rr[base + h * HD + k, :]
                    re = lax.bitcast_convert_type(lax.shift_left(ri, 16),
                                                  jnp.float32)
                    ro = lax.bitcast_convert_type(ri & jnp.int32(-65536),
                                                  jnp.float32)
                    acc0 = acc0 + wsp * re
                    acc1 = acc1 + wsp * ro
                oref = outL_v if h < 4 else outR_v
                oref[qi, pl.ds(hh * HD, 16)] = acc0
                oref[qi, pl.ds(hh * HD + 16, 16)] = acc1
            return c2

        lax.fori_loop(0, CQ, q_loop, 0)
        qs = chunk_q(c)
        pltpu.sync_copy(outL_v, outL_hbm.at[pl.ds(qs, CQ), :])
        pltpu.sync_copy(outR_v, outR_hbm.at[pl.ds(qs, CQ), :])

    start_io(0, 0)
    wait_io(0)
    fire_g(0)
    start_io(1, 1)

    def body3(i, carry):
        c = 3 * i
        wait_io(1); fire_g(1); start_io(c + 2, 2)
        wait_g(0); compute(c, 0)
        wait_io(2); fire_g(2); start_io(c + 3, 0)
        wait_g(1); compute(c + 1, 1)
        wait_io(0); fire_g(0); start_io(c + 4, 1)
        wait_g(2); compute(c + 2, 2)
        return carry

    lax.fori_loop(0, (NCHT - 2) // 3, body3, 0)
    wait_io(1)
    fire_g(1)
    wait_g(0)
    compute(NCHT - 2, 0)
    wait_g(1)
    compute(NCHT - 1, 1)


@functools.lru_cache(maxsize=1)
def _sc_call():
    return pl.kernel(
        _sc_body,
        out_type=[jax.ShapeDtypeStruct((NQ, 128), jnp.float32),
                  jax.ShapeDtypeStruct((NQ, 128), jnp.float32)],
        mesh=plsc.VectorSubcoreMesh(core_axis_name="c", subcore_axis_name="s"),
        scratch_types=(
            [pltpu.VMEM((CQ, 128), jnp.int32)] * 6
            + [pltpu.VMEM((CQ, 128), jnp.float32)] * 6
            + [pltpu.VMEM((ROWSC, HD // 2), jnp.int32)] * 3
            + [pltpu.VMEM((CQ, 128), jnp.float32)] * 2
            + [pltpu.SemaphoreType.DMA] * 6
        ),
        compiler_params=pltpu.CompilerParams(use_tc_tiling_on_sc=False),
    )


def _tc2_body(q_ref, sL_ref, sR_ref, WoL_ref, WoR_ref, bo_ref, o_ref):
    o_ref[...] = (q_ref[...] + bo_ref[...]
                  + jnp.dot(sL_ref[...], WoL_ref[...],
                            preferred_element_type=jnp.float32)
                  + jnp.dot(sR_ref[...], WoR_ref[...],
                            preferred_element_type=jnp.float32))


def _tc2_call(q, sL, sR, WoL, WoR, bo, interpret=False):
    return pl.pallas_call(
        _tc2_body,
        grid=(NQB,),
        in_specs=[
            pl.BlockSpec((QB, D), lambda i: (i, 0)),
            pl.BlockSpec((QB, 128), lambda i: (i, 0)),
            pl.BlockSpec((QB, 128), lambda i: (i, 0)),
            pl.BlockSpec((128, D), lambda i: (0, 0)),
            pl.BlockSpec((128, D), lambda i: (0, 0)),
            pl.BlockSpec((1, D), lambda i: (0, 0)),
        ],
        out_specs=pl.BlockSpec((QB, D), lambda i: (i, 0)),
        out_shape=jax.ShapeDtypeStruct((NQ, D), jnp.float32),
        interpret=interpret,
    )(q, sL, sR, WoL, WoR, bo)


def kernel(query, query_pos, reference_points, spatial_shapes,
           level_start_index, W_value, b_value, W_so, b_so, W_aw, b_aw,
           W_out, b_out):
    q2 = query[0]
    qp2 = query_pos[0]
    refx = reference_points[:, :, 0, 0].T
    refy = reference_points[:, :, 0, 1].T
    xb = refx[:, _BSEL] * 100.0 - 0.5
    yb = refy[:, _BSEL] * 100.0 - 0.5
    Wso_p = W_so[:, _PERM_SO]
    bso_p = b_so[_PERM_SO][None, :]
    Waw_p = W_aw[:, _PERM_AW]
    baw_p = b_aw[_PERM_AW][None, :]
    Wo_p = W_out[_PERM_WOUT, :]
    v, idxL, idxR, wL, wR = _tc1_call(q2, qp2, xb, yb, W_value,
                                      b_value[None, :], Wso_p, bso_p,
                                      Waw_p, baw_p)
    vtab = lax.bitcast_convert_type(
        v.reshape(NQ, 128, 2), jnp.int32).reshape(NQ * H, HD // 2)
    sL, sR = _sc_call()(vtab, idxL, idxR, wL, wR)
    out = _tc2_call(q2, sL, sR, Wo_p[:128], Wo_p[128:], b_out[None, :])
    return out[None]


# speedup vs baseline: 1.3517x; 1.3517x over previous
import functools

import jax
import jax.numpy as jnp
import numpy as np
from jax import lax
from jax.experimental import pallas as pl
from jax.experimental.pallas import tpu as pltpu
from jax.experimental.pallas import tpu_sc as plsc

D = 256
H = 8
HD = 32
NBQ = 2
NP = 4
GH, GW = 100, 100
NQ = GH * GW
RPQ = NBQ * NP * 4

QB = 400
NQB = NQ // QB

NW = 32
CQ = 8
ROWSC = CQ * D
NCHUNK = NQ // CQ
NCHT = 41

_PERM_SO = np.empty(128, np.int64)
_PERM_AW = np.empty(64, np.int64)
for _h in range(H):
    for _b in range(NBQ):
        for _p in range(NP):
            _s = _p * 16 + _h * 2 + _b
            _PERM_SO[_s] = _h * 16 + _b * 8 + _p * 2 + 0
            _PERM_SO[64 + _s] = _h * 16 + _b * 8 + _p * 2 + 1
            _PERM_AW[_s] = _h * 8 + _b * 4 + _p
_BSEL = np.arange(64) % 2

_S01 = np.zeros((64, 256), np.float32)
_S23 = np.zeros((64, 256), np.float32)
_SCW = [np.zeros((64, 256), np.float32) for _ in range(4)]
_BIAS_IDX = np.zeros((1, 256), np.float32)
for _t in range(256):
    _h = _t // 32
    _m = _t % 32
    _bp = _m // 4
    _c = _m % 4
    _b = _bp // 4
    _p = _bp % 4
    _s = _p * 16 + _h * 2 + _b
    if _c in (0, 1):
        _S01[_s, _t] = 1.0
    else:
        _S23[_s, _t] = 1.0
    _SCW[_c][_s, _t] = 1.0
    _BIAS_IDX[0, _t] = _h + (8.0 if _c in (1, 3) else 0.0)

_PEVEN = np.zeros((256, 128), np.float32)
_PODD = np.zeros((256, 128), np.float32)
for _j in range(128):
    _PEVEN[2 * _j, _j] = 1.0
    _PODD[2 * _j + 1, _j] = 1.0

_PERM_WOUT = np.empty(256, np.int64)
for _h in range(H):
    for _j in range(32):
        _ch = 2 * _j if _j < 16 else 2 * (_j - 16) + 1
        _PERM_WOUT[_h * 32 + _j] = _h * 32 + _ch


def _tc1_body(q_ref, qp_ref, xb_ref, yb_ref, Wv_ref, bv_ref, Wso_ref,
              bso_ref, Waw_ref, baw_ref, S01_ref, S23_ref, SC0_ref, SC1_ref,
              SC2_ref, SC3_ref, bi_ref, PE_ref, PO_ref, v_ref, idxL_ref,
              idxR_ref, wL_ref, wR_ref):
    q = q_ref[...]
    v = jnp.dot(q, Wv_ref[...], preferred_element_type=jnp.float32) + bv_ref[...]
    ve = jnp.dot(v, PE_ref[...], preferred_element_type=jnp.float32)
    vo = jnp.dot(v, PO_ref[...], preferred_element_type=jnp.float32)
    be = lax.bitcast_convert_type(
        ve.astype(jnp.bfloat16).astype(jnp.float32), jnp.int32)
    bo = lax.bitcast_convert_type(
        vo.astype(jnp.bfloat16).astype(jnp.float32), jnp.int32)
    v_ref[...] = bo | lax.shift_right_logical(be, 16)
    qc = jnp.concatenate([q, q + qp_ref[...]], axis=-1)
    so = jnp.dot(qc, Wso_ref[...], preferred_element_type=jnp.float32) + bso_ref[...]
    awl = jnp.dot(qc, Waw_ref[...], preferred_element_type=jnp.float32) + baw_ref[...]
    m = jnp.maximum(jnp.maximum(awl[:, 0:16], awl[:, 16:32]),
                    jnp.maximum(awl[:, 32:48], awl[:, 48:64]))
    e = jnp.exp(awl - jnp.concatenate([m, m, m, m], axis=-1))
    s = e[:, 0:16] + e[:, 16:32] + e[:, 32:48] + e[:, 48:64]
    aw = e / jnp.concatenate([s, s, s, s], axis=-1)

    xf = so[:, 0:64] + xb_ref[...]
    yf = so[:, 64:128] + yb_ref[...]
    x0 = jnp.floor(xf)
    fx = xf - x0
    cb = jnp.clip(x0, 0.0, 98.0)
    wc0 = jnp.where(x0 == cb, 1.0 - fx, jnp.where(x0 + 1.0 == cb, fx, 0.0))
    wc1 = jnp.where(x0 == cb, fx, jnp.where(x0 == cb + 1.0, 1.0 - fx, 0.0))
    y0 = jnp.floor(yf)
    fy = yf - y0
    rb0 = jnp.clip(y0, 0.0, 99.0)
    rb1 = jnp.clip(y0 + 1.0, 0.0, 99.0)
    wr0 = jnp.where(y0 == rb0, 1.0 - fy, 0.0)
    wr1 = jnp.where(y0 + 1.0 == rb1, fy, 0.0)
    idxA = rb0 * 100.0 + cb
    idxB = rb1 * 100.0 + cb
    idx256 = (jnp.dot(idxA, S01_ref[...], preferred_element_type=jnp.float32,
                      precision=lax.Precision.HIGHEST)
              + jnp.dot(idxB, S23_ref[...], preferred_element_type=jnp.float32,
                        precision=lax.Precision.HIGHEST)) * 8.0 + bi_ref[...]
    idx256 = (idx256 + 0.5).astype(jnp.int32)
    idxL_ref[...] = idx256[:, 0:128]
    idxR_ref[...] = idx256[:, 128:256]
    half = 0.5 * aw
    w256 = (
        jnp.dot(half * wr0 * wc0, SC0_ref[...], preferred_element_type=jnp.float32)
        + jnp.dot(half * wr0 * wc1, SC1_ref[...], preferred_element_type=jnp.float32)
        + jnp.dot(half * wr1 * wc0, SC2_ref[...], preferred_element_type=jnp.float32)
        + jnp.dot(half * wr1 * wc1, SC3_ref[...], preferred_element_type=jnp.float32))
    wL_ref[...] = w256[:, 0:128]
    wR_ref[...] = w256[:, 128:256]


def _tc1_call(q, qp, xb, yb, Wv, bv, Wso, bso, Waw, baw, interpret=False):
    consts = [jnp.asarray(a) for a in
              (_S01, _S23, *_SCW, _BIAS_IDX, _PEVEN, _PODD)]
    cspecs = ([pl.BlockSpec((64, 256), lambda i: (0, 0))] * 6
              + [pl.BlockSpec((1, 256), lambda i: (0, 0))]
              + [pl.BlockSpec((256, 128), lambda i: (0, 0))] * 2)
    h128 = pl.BlockSpec((QB, 128), lambda i: (i, 0))
    return pl.pallas_call(
        _tc1_body,
        grid=(NQB,),
        in_specs=[
            pl.BlockSpec((QB, D), lambda i: (i, 0)),
            pl.BlockSpec((QB, D), lambda i: (i, 0)),
            pl.BlockSpec((QB, 64), lambda i: (i, 0)),
            pl.BlockSpec((QB, 64), lambda i: (i, 0)),
            pl.BlockSpec((D, D), lambda i: (0, 0)),
            pl.BlockSpec((1, D), lambda i: (0, 0)),
            pl.BlockSpec((2 * D, 128), lambda i: (0, 0)),
            pl.BlockSpec((1, 128), lambda i: (0, 0)),
            pl.BlockSpec((2 * D, 64), lambda i: (0, 0)),
            pl.BlockSpec((1, 64), lambda i: (0, 0)),
        ] + cspecs,
        out_specs=[h128, h128, h128, h128, h128],
        out_shape=[
            jax.ShapeDtypeStruct((NQ, 128), jnp.int32),
            jax.ShapeDtypeStruct((NQ, 128), jnp.int32),
            jax.ShapeDtypeStruct((NQ, 128), jnp.int32),
            jax.ShapeDtypeStruct((NQ, 128), jnp.float32),
            jax.ShapeDtypeStruct((NQ, 128), jnp.float32),
        ],
        interpret=interpret,
    )(q, qp, xb, yb, Wv, bv, Wso, bso, Waw, baw, *consts)


def _splat(vec, lane):
    return lax.gather(
        vec, jnp.full((16, 1), lane, jnp.int32),
        dimension_numbers=lax.GatherDimensionNumbers(
            offset_dims=(), collapsed_slice_dims=(0,), start_index_map=(0,)),
        slice_sizes=(1,),
        mode=lax.GatherScatterMode.PROMISE_IN_BOUNDS)


def _sc_body(vt_hbm, idxL_hbm, idxR_hbm, wL_hbm, wR_hbm, outL_hbm, outR_hbm,
             iL0, iL1, iL2, iR0, iR1, iR2, wl0, wl1, wl2, wr0, wr1, wr2,
             r0, r1, r2, outL_v, outR_v,
             sio0, sio1, sio2, sg0, sg1, sg2):
    idxLs = (iL0, iL1, iL2)
    idxRs = (iR0, iR1, iR2)
    wLs = (wl0, wl1, wl2)
    wRs = (wr0, wr1, wr2)
    rows = (r0, r1, r2)
    sio = (sio0, sio1, sio2)
    sg = (sg0, sg1, sg2)
    wid = lax.axis_index("s") * 2 + lax.axis_index("c")
    c0 = (625 * wid) // 16

    def chunk_q(c):
        return jnp.minimum(c0 + c, NCHUNK - 1) * CQ

    def start_io(c, b):
        qs = chunk_q(c)
        pltpu.async_copy(idxL_hbm.at[pl.ds(qs, CQ), :], idxLs[b], sio[b])
        pltpu.async_copy(idxR_hbm.at[pl.ds(qs, CQ), :], idxRs[b], sio[b])
        pltpu.async_copy(wL_hbm.at[pl.ds(qs, CQ), :], wLs[b], sio[b])
        pltpu.async_copy(wR_hbm.at[pl.ds(qs, CQ), :], wRs[b], sio[b])

    def wait_io(b):
        pltpu.make_async_copy(idxL_hbm.at[pl.ds(0, CQ), :], idxLs[b], sio[b]).wait()
        pltpu.make_async_copy(idxR_hbm.at[pl.ds(0, CQ), :], idxRs[b], sio[b]).wait()
        pltpu.make_async_copy(wL_hbm.at[pl.ds(0, CQ), :], wLs[b], sio[b]).wait()
        pltpu.make_async_copy(wR_hbm.at[pl.ds(0, CQ), :], wRs[b], sio[b]).wait()

    def fire_g(b):
        for qi in range(CQ):
            pltpu.async_copy(
                vt_hbm.at[idxLs[b].at[qi, :]],
                rows[b].at[pl.ds(qi * 2 * 128, 128)], sg[b])
            pltpu.async_copy(
                vt_hbm.at[idxRs[b].at[qi, :]],
                rows[b].at[pl.ds((qi * 2 + 1) * 128, 128)], sg[b])

    def wait_g(b):
        for qi in range(CQ):
            pltpu.make_async_copy(
                vt_hbm.at[idxLs[b].at[qi, :]],
                rows[b].at[pl.ds(qi * 2 * 128, 128)], sg[b]).wait()
            pltpu.make_async_copy(
                vt_hbm.at[idxRs[b].at[qi, :]],
                rows[b].at[pl.ds((qi * 2 + 1) * 128, 128)], sg[b]).wait()

    def compute(c, b):
        rr = rows[b]
        wl = wLs[b]
        wr = wRs[b]

        def q_loop(qi, c2):
            base = qi * D
            for h in range(H):
                wref = wl if h < 4 else wr
                hh = h % 4
                wv0 = wref[qi, pl.ds(hh * HD, 16)]
                wv1 = wref[qi, pl.ds(hh * HD + 16, 16)]
                acc0 = jnp.zeros((16,), jnp.float32)
                acc1 = jnp.zeros((16,), jnp.float32)
                for k in range(RPQ):
                    wsp = _splat(wv0 if k < 16 else wv1, k % 16)
                    ri = rr[base + h * HD + k, :]
                    re = lax.bitcast_convert_type(lax.shift_left(ri, 16),
                                                  jnp.float32)
                    ro = lax.bitcast_convert_type(ri & jnp.int32(-65536),
                                                  jnp.float32)
                    acc0 = acc0 + wsp * re
                    acc1 = acc1 + wsp * ro
                oref = outL_v if h < 4 else outR_v
                oref[qi, pl.ds(hh * HD, 16)] = acc0
                oref[qi, pl.ds(hh * HD + 16, 16)] = acc1
            return c2

        lax.fori_loop(0, CQ, q_loop, 0)
        qs = chunk_q(c)
        pltpu.sync_copy(outL_v, outL_hbm.at[pl.ds(qs, CQ), :])
        pltpu.sync_copy(outR_v, outR_hbm.at[pl.ds(qs, CQ), :])

    start_io(0, 0)
    wait_io(0)
    fire_g(0)
    start_io(1, 1)

    def body3(i, carry):
        c = 3 * i
        wait_io(1); fire_g(1); start_io(c + 2, 2)
        wait_g(0); compute(c, 0)
        wait_io(2); fire_g(2); start_io(c + 3, 0)
        wait_g(1); compute(c + 1, 1)
        wait_io(0); fire_g(0); start_io(c + 4, 1)
        wait_g(2); compute(c + 2, 2)
        return carry

    lax.fori_loop(0, (NCHT - 2) // 3, body3, 0)
    wait_io(1)
    fire_g(1)
    wait_g(0)
    compute(NCHT - 2, 0)
    wait_g(1)
    compute(NCHT - 1, 1)


@functools.lru_cache(maxsize=1)
def _sc_call():
    return pl.kernel(
        _sc_body,
        out_type=[jax.ShapeDtypeStruct((NQ, 128), jnp.float32),
                  jax.ShapeDtypeStruct((NQ, 128), jnp.float32)],
        mesh=plsc.VectorSubcoreMesh(core_axis_name="c", subcore_axis_name="s"),
        scratch_types=(
            [pltpu.VMEM((CQ, 128), jnp.int32)] * 6
            + [pltpu.VMEM((CQ, 128), jnp.float32)] * 6
            + [pltpu.VMEM((ROWSC, HD // 2), jnp.int32)] * 3
            + [pltpu.VMEM((CQ, 128), jnp.float32)] * 2
            + [pltpu.SemaphoreType.DMA] * 6
        ),
        compiler_params=pltpu.CompilerParams(use_tc_tiling_on_sc=False),
    )


def _tc2_body(q_ref, sL_ref, sR_ref, WoL_ref, WoR_ref, bo_ref, o_ref):
    o_ref[...] = (q_ref[...] + bo_ref[...]
                  + jnp.dot(sL_ref[...], WoL_ref[...],
                            preferred_element_type=jnp.float32)
                  + jnp.dot(sR_ref[...], WoR_ref[...],
                            preferred_element_type=jnp.float32))


def _tc2_call(q, sL, sR, WoL, WoR, bo, interpret=False):
    return pl.pallas_call(
        _tc2_body,
        grid=(NQB,),
        in_specs=[
            pl.BlockSpec((QB, D), lambda i: (i, 0)),
            pl.BlockSpec((QB, 128), lambda i: (i, 0)),
            pl.BlockSpec((QB, 128), lambda i: (i, 0)),
            pl.BlockSpec((128, D), lambda i: (0, 0)),
            pl.BlockSpec((128, D), lambda i: (0, 0)),
            pl.BlockSpec((1, D), lambda i: (0, 0)),
        ],
        out_specs=pl.BlockSpec((QB, D), lambda i: (i, 0)),
        out_shape=jax.ShapeDtypeStruct((NQ, D), jnp.float32),
        interpret=interpret,
    )(q, sL, sR, WoL, WoR, bo)


def kernel(query, query_pos, reference_points, spatial_shapes,
           level_start_index, W_value, b_value, W_so, b_so, W_aw, b_aw,
           W_out, b_out):
    q2 = query[0]
    qp2 = query_pos[0]
    refx = reference_points[:, :, 0, 0].T
    refy = reference_points[:, :, 0, 1].T
    xb = refx[:, _BSEL] * 100.0 - 0.5
    yb = refy[:, _BSEL] * 100.0 - 0.5
    Wso_p = W_so[:, _PERM_SO]
    bso_p = b_so[_PERM_SO][None, :]
    Waw_p = W_aw[:, _PERM_AW]
    baw_p = b_aw[_PERM_AW][None, :]
    Wo_p = W_out[_PERM_WOUT, :]
    v, idxL, idxR, wL, wR = _tc1_call(q2, qp2, xb, yb, W_value,
                                      b_value[None, :], Wso_p, bso_p,
                                      Waw_p, baw_p)
    vtab = v.reshape(NQ * H, HD // 2)
    sL, sR = _sc_call()(vtab, idxL, idxR, wL, wR)
    out = _tc2_call(q2, sL, sR, Wo_p[:128], Wo_p[128:], b_out[None, :])
    return out[None]
